# baseline (speedup 1.0000x reference)
import jax
import jax.numpy as jnp
from jax import lax
from jax.experimental import pallas as pl
from jax.experimental.pallas import tpu as pltpu
from jax.experimental.pallas import tpu_sc as plsc

N = 10000
NPAD = 10112
D = 128
H = 64
NC = 2
NS = 16
NW = NC * NS
CH = 128
BLK = 16
NBUF = 4
DEGCH = 128
RPT = NPAD // NS
RT = 2000
STAGE = [128, 128, 128, 128, 120]


def _sc_mesh():
    return plsc.VectorSubcoreMesh(core_axis_name="c", subcore_axis_name="s")


def _deg_body(dst_hbm, zeros_hbm, ones_hbm, deg_hbm, idx_v, ones_v, stage_v,
              acc_sh):
    c = lax.axis_index("c")
    s = lax.axis_index("s")
    nch = dst_hbm.shape[1]
    w = c * NS + s

    pltpu.sync_copy(zeros_hbm, stage_v)
    pltpu.sync_copy(stage_v, acc_sh.at[pl.ds(s * RPT, RPT)])
    pltpu.sync_copy(ones_hbm, ones_v)
    pltpu.sync_copy(dst_hbm.at[w], idx_v)
    plsc.subcore_barrier()

    def body(j, carry):
        pltpu.sync_copy(ones_v, acc_sh.at[idx_v.at[j]], add=True)
        return carry

    lax.fori_loop(0, nch, body, 0)
    plsc.subcore_barrier()
    pltpu.sync_copy(acc_sh.at[pl.ds(s * RPT, RPT)], stage_v)
    pltpu.sync_copy(stage_v, deg_hbm.at[c, pl.ds(s * RPT, RPT)])


def _deg_call(dst_r, zeros16, ones16):
    nch = dst_r.shape[1]
    return pl.kernel(
        _deg_body,
        out_type=jax.ShapeDtypeStruct((NC, NPAD, 16), jnp.float32),
        mesh=_sc_mesh(),
        scratch_types=[
            pltpu.VMEM((nch, DEGCH), jnp.int32),
            pltpu.VMEM((DEGCH, 16), jnp.float32),
            pltpu.VMEM((RPT, 16), jnp.float32),
            pltpu.VMEM_SHARED((NPAD, 16), jnp.float32),
        ],
        compiler_params=pltpu.CompilerParams(use_tc_tiling_on_sc=False),
    )(dst_r, zeros16, ones16)


def _agg_body(hs_hbm, src_hbm, dst_hbm, out_hbm, rsrc_v, rdst_v,
              gb0, gb1, gb2, gb3, gs0, gs1, gs2, gs3, ss0, ss1, ss2, ss3,
              rs0, rs1, rd0, rd1, table_sh, acc_sh):
    c = lax.axis_index("c")
    s = lax.axis_index("s")
    nch = dst_hbm.shape[1]
    nblk = nch // BLK
    r0 = s * RPT

    off = 0
    for sz in STAGE:
        rr = r0 + off
        pltpu.sync_copy(hs_hbm.at[c, pl.ds(rr, sz)], gb0.at[pl.ds(0, sz)])
        pltpu.sync_copy(gb0.at[pl.ds(0, sz)], table_sh.at[pl.ds(rr, sz)])
        pltpu.sync_copy(gb0.at[pl.ds(0, sz)], acc_sh.at[pl.ds(rr, sz)])
        off += sz

    rsems = (rs0, rs1)
    dsems = (rd0, rd1)

    def refill_src(blk, start=True):
        h = blk % 2
        cp = pltpu.make_async_copy(src_hbm.at[s, pl.ds(blk * BLK, BLK)],
                                   rsrc_v.at[h], rsems[h])
        cp.start() if start else cp.wait()

    def refill_dst(blk, start=True):
        h = blk % 2
        cp = pltpu.make_async_copy(dst_hbm.at[s, pl.ds(blk * BLK, BLK)],
                                   rdst_v.at[h], dsems[h])
        cp.start() if start else cp.wait()

    pltpu.sync_copy(src_hbm.at[s, pl.ds(0, BLK)], rsrc_v.at[0])
    pltpu.sync_copy(dst_hbm.at[s, pl.ds(0, BLK)], rdst_v.at[0])
    refill_src(1)
    refill_dst(1)
    plsc.subcore_barrier()

    bufs = (gb0, gb1, gb2, gb3)
    gsems = (gs0, gs1, gs2, gs3)
    ssems = (ss0, ss1, ss2, ss3)

    def g_start(h, r, b):
        pltpu.async_copy(table_sh.at[rsrc_v.at[h, r]], bufs[b], gsems[b])

    def g_wait(h, r, b):
        pltpu.make_async_copy(table_sh.at[rsrc_v.at[h, r]], bufs[b],
                              gsems[b]).wait()

    def s_start(h, r, b):
        pltpu.async_copy(bufs[b], acc_sh.at[rdst_v.at[h, r]], ssems[b],
                         add=True)

    def s_wait(h, r, b):
        pltpu.make_async_copy(bufs[b], acc_sh.at[rdst_v.at[h, r]],
                              ssems[b]).wait()

    for b in range(NBUF):
        g_start(0, b, b)

    for bi in range(nblk):
        h = bi % 2
        hn = (bi + 1) % 2
        refill_src(bi + 1, start=False)
        if bi + 1 < nblk:
            refill_dst(bi + 1, start=False)

        def group(kk, carry):
            r = NBUF * kk
            for b in range(NBUF):
                g_wait(h, r + b, b)
                s_start(h, r + b, b)
            for b in range(NBUF):
                s_wait(h, r + b, b)
                g_start(h, r + NBUF + b, b)
            return carry

        lax.fori_loop(0, BLK // NBUF - 1, group, 0)
        r = BLK - NBUF
        for b in range(NBUF):
            g_wait(h, r + b, b)
            s_start(h, r + b, b)
        for b in range(NBUF):
            s_wait(h, r + b, b)
            g_start(hn, b, b)

        if bi + 2 <= nblk:
            refill_src(bi + 2)
        if bi + 2 < nblk:
            refill_dst(bi + 2)

    for b in range(NBUF):
        g_wait(nblk % 2, b, b)

    plsc.subcore_barrier()
    off = 0
    for sz in STAGE:
        rr = r0 + off
        pltpu.sync_copy(acc_sh.at[pl.ds(rr, sz)], gb0.at[pl.ds(0, sz)])
        pltpu.sync_copy(gb0.at[pl.ds(0, sz)], out_hbm.at[c, pl.ds(rr, sz)])
        off += sz


def _agg_call(hs, srcd_r, dst_r):
    return pl.kernel(
        _agg_body,
        out_type=jax.ShapeDtypeStruct((NC, NPAD, H), jnp.float32),
        mesh=_sc_mesh(),
        scratch_types=[
            pltpu.VMEM((2, BLK, CH), jnp.int32),
            pltpu.VMEM((2, BLK, CH), jnp.int32),
            pltpu.VMEM((CH, H), jnp.float32),
            pltpu.VMEM((CH, H), jnp.float32),
            pltpu.VMEM((CH, H), jnp.float32),
            pltpu.VMEM((CH, H), jnp.float32),
            pltpu.SemaphoreType.DMA,
            pltpu.SemaphoreType.DMA,
            pltpu.SemaphoreType.DMA,
            pltpu.SemaphoreType.DMA,
            pltpu.SemaphoreType.DMA,
            pltpu.SemaphoreType.DMA,
            pltpu.SemaphoreType.DMA,
            pltpu.SemaphoreType.DMA,
            pltpu.SemaphoreType.DMA,
            pltpu.SemaphoreType.DMA,
            pltpu.SemaphoreType.DMA,
            pltpu.SemaphoreType.DMA,
            pltpu.VMEM_SHARED((NPAD, H), jnp.float32),
            pltpu.VMEM_SHARED((NPAD, H), jnp.float32),
        ],
        compiler_params=pltpu.CompilerParams(use_tc_tiling_on_sc=False),
    )(hs, srcd_r, dst_r)


def _mm_body(x_ref, w_ref, deg_ref, o_ref):
    deg = deg_ref[0, :, 0:1] + deg_ref[1, :, 0:1]
    dinv = lax.rsqrt(deg + 1.0)
    o_ref[0] = dinv * jnp.dot(x_ref[...], w_ref[0],
                              preferred_element_type=jnp.float32)


def _mm_call(x, w_split, deg):
    return pl.pallas_call(
        _mm_body,
        grid=(N // RT, NC),
        in_specs=[
            pl.BlockSpec((RT, D), lambda i, j: (i, 0)),
            pl.BlockSpec((1, D, H), lambda i, j: (j, 0, 0)),
            pl.BlockSpec((NC, RT, 16), lambda i, j: (0, i, 0)),
        ],
        out_specs=pl.BlockSpec((1, RT, H), lambda i, j: (j, i, 0)),
        out_shape=jax.ShapeDtypeStruct((NC, NPAD, H), jnp.float32),
    )(x, w_split, deg)


def _postmm_body(a_ref, deg_ref, b_ref, w_ref, o_ref):
    deg = deg_ref[0, :, 0:1] + deg_ref[1, :, 0:1]
    dinv = lax.rsqrt(deg + 1.0)
    z0 = jnp.maximum(dinv * a_ref[0] + b_ref[0], 0.0)
    z1 = jnp.maximum(dinv * a_ref[1] + b_ref[1], 0.0)
    z = jnp.concatenate([z0, z1], axis=-1)
    o_ref[0] = dinv * jnp.dot(z, w_ref[0],
                              preferred_element_type=jnp.float32)


def _postmm_call(a, deg, b_split, w_split):
    return pl.pallas_call(
        _postmm_body,
        grid=(N // RT, NC),
        in_specs=[
            pl.BlockSpec((NC, RT, H), lambda i, j: (0, i, 0)),
            pl.BlockSpec((NC, RT, 16), lambda i, j: (0, i, 0)),
            pl.BlockSpec((NC, 1, H), lambda i, j: (0, 0, 0)),
            pl.BlockSpec((1, D, H), lambda i, j: (j, 0, 0)),
        ],
        out_specs=pl.BlockSpec((1, RT, H), lambda i, j: (j, i, 0)),
        out_shape=jax.ShapeDtypeStruct((NC, NPAD, H), jnp.float32),
    )(a, deg, b_split, w_split)


def _posthead_body(a_ref, deg_ref, b_ref, w1_ref, b1_ref, pa_ref, w2_ref,
                   b2_ref, z_ref, p_ref):
    deg = deg_ref[0, :, 0:1] + deg_ref[1, :, 0:1]
    dinv = lax.rsqrt(deg + 1.0)
    z0 = jnp.maximum(dinv * a_ref[0] + b_ref[0], 0.0)
    z1 = jnp.maximum(dinv * a_ref[1] + b_ref[1], 0.0)
    z = jnp.concatenate([z0, z1], axis=-1)
    z_ref[...] = z
    p = jnp.dot(z, w1_ref[...],
                preferred_element_type=jnp.float32) + b1_ref[...]
    p = jnp.where(p > 0, p, pa_ref[0, 0] * p)
    p_ref[...] = jnp.dot(p, w2_ref[...],
                         preferred_element_type=jnp.float32) + b2_ref[...]


def _posthead_call(a, deg, b_split, w1, b1, pa, w2, b2):
    full = lambda i: (0, 0)
    return pl.pallas_call(
        _posthead_body,
        grid=(N // RT,),
        in_specs=[
            pl.BlockSpec((NC, RT, H), lambda i: (0, i, 0)),
            pl.BlockSpec((NC, RT, 16), lambda i: (0, i, 0)),
            pl.BlockSpec((NC, 1, H), lambda i: (0, 0, 0)),
            pl.BlockSpec((D, D), full),
            pl.BlockSpec((1, D), full),
            pl.BlockSpec((1, 1), full),
            pl.BlockSpec((D, D), full),
            pl.BlockSpec((1, D), full),
        ],
        out_specs=(
            pl.BlockSpec((RT, D), lambda i: (i, 0)),
            pl.BlockSpec((RT, D), lambda i: (i, 0)),
        ),
        out_shape=(
            jax.ShapeDtypeStruct((N, D), jnp.float32),
            jax.ShapeDtypeStruct((N, D), jnp.float32),
        ),
    )(a, deg, b_split, w1, b1, pa, w2, b2)


def kernel(x, edge_index, W1, b1, W2, b2, Wp1, bp1, prelu_a, Wp2, bp2):
    E = edge_index.shape[1]
    src = edge_index[0]
    dst = edge_index[1]

    nch = -(-E // (NS * CH))
    nch = -(-nch // BLK) * BLK
    ep = NS * CH * nch - E
    pad_ids = jnp.arange(ep, dtype=jnp.int32)
    src_r = jnp.concatenate([src, pad_ids % 256]).reshape(NS, nch, CH)
    dst_r = jnp.concatenate([dst, N + (pad_ids % (NPAD - N))]
                            ).reshape(NS, nch, CH)
    dummy = (jnp.arange(NS * BLK * CH, dtype=jnp.int32) % 256
             ).reshape(NS, BLK, CH)
    srcd_r = jnp.concatenate([src_r, dummy], axis=1)

    nchd = -(-E // (NW * DEGCH))
    epd = NW * DEGCH * nchd - E
    padd = jnp.arange(epd, dtype=jnp.int32)
    dstd_r = jnp.concatenate([dst, N + (padd % (NPAD - N))]
                             ).reshape(NW, nchd, DEGCH)

    zeros16 = jnp.zeros((RPT, 16), jnp.float32)
    ones16 = jnp.ones((DEGCH, 16), jnp.float32)

    W1s = jnp.stack([W1[:, :H], W1[:, H:]])
    W2s = jnp.stack([W2[:, :H], W2[:, H:]])
    b1s = b1.reshape(NC, 1, H)
    b2s = b2.reshape(NC, 1, H)

    deg = _deg_call(dstd_r, zeros16, ones16)
    hs1 = _mm_call(x, W1s, deg)
    a1 = _agg_call(hs1, srcd_r, dst_r)
    hs2 = _postmm_call(a1, deg, b1s, W2s)
    a2 = _agg_call(hs2, srcd_r, dst_r)
    z2, p = _posthead_call(a2, deg, b2s, Wp1, bp1.reshape(1, D),
                           prelu_a.reshape(1, 1), Wp2, bp2.reshape(1, D))
    return (z2, p)


# speedup vs baseline: 1.3137x; 1.3137x over previous
import jax
import jax.numpy as jnp
from jax import lax
from jax.experimental import pallas as pl
from jax.experimental.pallas import tpu as pltpu
from jax.experimental.pallas import tpu_sc as plsc

N = 10000
NPAD = 10112
D = 128
NC = 2
NS = 16
NW = NC * NS
CH = 128
BLK = 16
GSPLIT = 4
DEGCH = 128
RPT = NPAD // NS
RT = 2000
STAGE = [128, 128, 128, 128, 120]


def _sc_mesh():
    return plsc.VectorSubcoreMesh(core_axis_name="c", subcore_axis_name="s")


def _deg_body(dst_hbm, zeros_hbm, ones_hbm, deg_hbm, idx_v, ones_v, stage_v,
              acc_sh):
    c = lax.axis_index("c")
    s = lax.axis_index("s")
    nch = dst_hbm.shape[1]
    w = c * NS + s

    pltpu.sync_copy(zeros_hbm, stage_v)
    pltpu.sync_copy(stage_v, acc_sh.at[pl.ds(s * RPT, RPT)])
    pltpu.sync_copy(ones_hbm, ones_v)
    pltpu.sync_copy(dst_hbm.at[w], idx_v)
    plsc.subcore_barrier()

    def body(j, carry):
        pltpu.sync_copy(ones_v, acc_sh.at[idx_v.at[j]], add=True)
        return carry

    lax.fori_loop(0, nch, body, 0)
    plsc.subcore_barrier()
    pltpu.sync_copy(acc_sh.at[pl.ds(s * RPT, RPT)], stage_v)
    pltpu.sync_copy(stage_v, deg_hbm.at[c, pl.ds(s * RPT, RPT)])


def _deg_call(dst_r, zeros16, ones16):
    nch = dst_r.shape[1]
    return pl.kernel(
        _deg_body,
        out_type=jax.ShapeDtypeStruct((NC, NPAD, 16), jnp.float32),
        mesh=_sc_mesh(),
        scratch_types=[
            pltpu.VMEM((nch, DEGCH), jnp.int32),
            pltpu.VMEM((DEGCH, 16), jnp.float32),
            pltpu.VMEM((RPT, 16), jnp.float32),
            pltpu.VMEM_SHARED((NPAD, 16), jnp.float32),
        ],
        compiler_params=pltpu.CompilerParams(use_tc_tiling_on_sc=False),
    )(dst_r, zeros16, ones16)


def _agg_body(hs_hbm, src_hbm, dst_hbm, zeros_hbm, out_hbm, rsrc_v, rdst_v,
              gb0, gb1, gs0, gs1, ss0, ss1, rs0, rs1, rd0, rd1, acc_sh):
    c = lax.axis_index("c")
    s = lax.axis_index("s")
    nch = dst_hbm.shape[1]
    nblk = nch // BLK
    w = c * NS + s
    r0 = s * RPT

    off = 0
    for sz in STAGE:
        rr = r0 + off

        @pl.when(c == 0)
        def _():
            pltpu.sync_copy(hs_hbm.at[pl.ds(rr, sz)], gb0.at[pl.ds(0, sz)])

        @pl.when(c != 0)
        def _():
            pltpu.sync_copy(zeros_hbm.at[pl.ds(0, sz)], gb0.at[pl.ds(0, sz)])

        pltpu.sync_copy(gb0.at[pl.ds(0, sz)], acc_sh.at[pl.ds(rr, sz)])
        off += sz

    rsems = (rs0, rs1)
    dsems = (rd0, rd1)

    def refill_src(blk, start=True):
        h = blk % 2
        cp = pltpu.make_async_copy(src_hbm.at[w, pl.ds(blk * BLK, BLK)],
                                   rsrc_v.at[h], rsems[h])
        cp.start() if start else cp.wait()

    def refill_dst(blk, start=True):
        h = blk % 2
        cp = pltpu.make_async_copy(dst_hbm.at[w, pl.ds(blk * BLK, BLK)],
                                   rdst_v.at[h], dsems[h])
        cp.start() if start else cp.wait()

    pltpu.sync_copy(src_hbm.at[w, pl.ds(0, BLK)], rsrc_v.at[0])
    pltpu.sync_copy(dst_hbm.at[w, pl.ds(0, BLK)], rdst_v.at[0])
    refill_src(1)
    refill_dst(1)
    plsc.subcore_barrier()

    QC = CH // GSPLIT

    def g_start(h, r, buf, sem):
        for q in range(GSPLIT):
            pltpu.async_copy(hs_hbm.at[rsrc_v.at[h, r, pl.ds(q * QC, QC)]],
                             buf.at[pl.ds(q * QC, QC)], sem)

    def g_wait(h, r, buf, sem):
        for q in range(GSPLIT):
            pltpu.make_async_copy(
                hs_hbm.at[rsrc_v.at[h, r, pl.ds(q * QC, QC)]],
                buf.at[pl.ds(q * QC, QC)], sem).wait()

    def s_start(h, r, buf, sem):
        pltpu.async_copy(buf, acc_sh.at[rdst_v.at[h, r]], sem, add=True)

    def s_wait(h, r, buf, sem):
        pltpu.make_async_copy(buf, acc_sh.at[rdst_v.at[h, r]], sem).wait()

    g_start(0, 0, gb0, gs0)
    g_start(0, 1, gb1, gs1)

    for bi in range(nblk):
        h = bi % 2
        hn = (bi + 1) % 2
        refill_src(bi + 1, start=False)
        if bi + 1 < nblk:
            refill_dst(bi + 1, start=False)

        def pair(kk, carry):
            r = 2 * kk
            g_wait(h, r, gb0, gs0)
            s_start(h, r, gb0, ss0)
            g_wait(h, r + 1, gb1, gs1)
            s_start(h, r + 1, gb1, ss1)
            s_wait(h, r, gb0, ss0)
            g_start(h, r + 2, gb0, gs0)
            s_wait(h, r + 1, gb1, ss1)
            g_start(h, r + 3, gb1, gs1)
            return carry

        lax.fori_loop(0, BLK // 2 - 1, pair, 0)
        r = BLK - 2
        g_wait(h, r, gb0, gs0)
        s_start(h, r, gb0, ss0)
        g_wait(h, r + 1, gb1, gs1)
        s_start(h, r + 1, gb1, ss1)
        s_wait(h, r, gb0, ss0)
        g_start(hn, 0, gb0, gs0)
        s_wait(h, r + 1, gb1, ss1)
        g_start(hn, 1, gb1, gs1)

        if bi + 2 <= nblk:
            refill_src(bi + 2)
        if bi + 2 < nblk:
            refill_dst(bi + 2)

    g_wait(nblk % 2, 0, gb0, gs0)
    g_wait(nblk % 2, 1, gb1, gs1)

    plsc.subcore_barrier()
    off = 0
    for sz in STAGE:
        rr = r0 + off
        pltpu.sync_copy(acc_sh.at[pl.ds(rr, sz)], gb0.at[pl.ds(0, sz)])
        pltpu.sync_copy(gb0.at[pl.ds(0, sz)], out_hbm.at[c, pl.ds(rr, sz)])
        off += sz


def _agg_call(hs, srcd_r, dst_r, zeros128):
    return pl.kernel(
        _agg_body,
        out_type=jax.ShapeDtypeStruct((NC, NPAD, D), jnp.float32),
        mesh=_sc_mesh(),
        scratch_types=[
            pltpu.VMEM((2, BLK, CH), jnp.int32),
            pltpu.VMEM((2, BLK, CH), jnp.int32),
            pltpu.VMEM((CH, D), jnp.float32),
            pltpu.VMEM((CH, D), jnp.float32),
            pltpu.SemaphoreType.DMA,
            pltpu.SemaphoreType.DMA,
            pltpu.SemaphoreType.DMA,
            pltpu.SemaphoreType.DMA,
            pltpu.SemaphoreType.DMA,
            pltpu.SemaphoreType.DMA,
            pltpu.SemaphoreType.DMA,
            pltpu.SemaphoreType.DMA,
            pltpu.VMEM_SHARED((NPAD, D), jnp.float32),
        ],
    )(hs, srcd_r, dst_r, zeros128)


def _mm_body(x_ref, w_ref, deg_ref, o_ref):
    deg = deg_ref[0, :, 0:1] + deg_ref[1, :, 0:1]
    dinv = lax.rsqrt(deg + 1.0)
    o_ref[...] = dinv * jnp.dot(x_ref[...], w_ref[...],
                                preferred_element_type=jnp.float32)


def _mm_call(x, w, deg):
    return pl.pallas_call(
        _mm_body,
        grid=(N // RT,),
        in_specs=[
            pl.BlockSpec((RT, D), lambda i: (i, 0)),
            pl.BlockSpec((D, D), lambda i: (0, 0)),
            pl.BlockSpec((NC, RT, 16), lambda i: (0, i, 0)),
        ],
        out_specs=pl.BlockSpec((RT, D), lambda i: (i, 0)),
        out_shape=jax.ShapeDtypeStruct((NPAD, D), jnp.float32),
    )(x, w, deg)


def _postmm_body(a_ref, deg_ref, b_ref, w_ref, o_ref):
    deg = deg_ref[0, :, 0:1] + deg_ref[1, :, 0:1]
    dinv = lax.rsqrt(deg + 1.0)
    z = jnp.maximum(dinv * (a_ref[0] + a_ref[1]) + b_ref[...], 0.0)
    o_ref[...] = dinv * jnp.dot(z, w_ref[...],
                                preferred_element_type=jnp.float32)


def _postmm_call(a, deg, b, w):
    return pl.pallas_call(
        _postmm_body,
        grid=(N // RT,),
        in_specs=[
            pl.BlockSpec((NC, RT, D), lambda i: (0, i, 0)),
            pl.BlockSpec((NC, RT, 16), lambda i: (0, i, 0)),
            pl.BlockSpec((1, D), lambda i: (0, 0)),
            pl.BlockSpec((D, D), lambda i: (0, 0)),
        ],
        out_specs=pl.BlockSpec((RT, D), lambda i: (i, 0)),
        out_shape=jax.ShapeDtypeStruct((NPAD, D), jnp.float32),
    )(a, deg, b, w)


def _posthead_body(a_ref, deg_ref, b_ref, w1_ref, b1_ref, pa_ref, w2_ref,
                   b2_ref, z_ref, p_ref):
    deg = deg_ref[0, :, 0:1] + deg_ref[1, :, 0:1]
    dinv = lax.rsqrt(deg + 1.0)
    z = jnp.maximum(dinv * (a_ref[0] + a_ref[1]) + b_ref[...], 0.0)
    z_ref[...] = z
    p = jnp.dot(z, w1_ref[...],
                preferred_element_type=jnp.float32) + b1_ref[...]
    p = jnp.where(p > 0, p, pa_ref[0, 0] * p)
    p_ref[...] = jnp.dot(p, w2_ref[...],
                         preferred_element_type=jnp.float32) + b2_ref[...]


def _posthead_call(a, deg, b, w1, b1, pa, w2, b2):
    full = lambda i: (0, 0)
    return pl.pallas_call(
        _posthead_body,
        grid=(N // RT,),
        in_specs=[
            pl.BlockSpec((NC, RT, D), lambda i: (0, i, 0)),
            pl.BlockSpec((NC, RT, 16), lambda i: (0, i, 0)),
            pl.BlockSpec((1, D), full),
            pl.BlockSpec((D, D), full),
            pl.BlockSpec((1, D), full),
            pl.BlockSpec((1, 1), full),
            pl.BlockSpec((D, D), full),
            pl.BlockSpec((1, D), full),
        ],
        out_specs=(
            pl.BlockSpec((RT, D), lambda i: (i, 0)),
            pl.BlockSpec((RT, D), lambda i: (i, 0)),
        ),
        out_shape=(
            jax.ShapeDtypeStruct((N, D), jnp.float32),
            jax.ShapeDtypeStruct((N, D), jnp.float32),
        ),
    )(a, deg, b, w1, b1, pa, w2, b2)


def kernel(x, edge_index, W1, b1, W2, b2, Wp1, bp1, prelu_a, Wp2, bp2):
    E = edge_index.shape[1]
    src = edge_index[0]
    dst = edge_index[1]

    nch = -(-E // (NW * CH))
    nch = -(-nch // BLK) * BLK
    ep = NW * CH * nch - E
    pad_ids = jnp.arange(ep, dtype=jnp.int32)
    src_r = jnp.concatenate([src, pad_ids % 256]).reshape(NW, nch, CH)
    dst_r = jnp.concatenate([dst, N + (pad_ids % (NPAD - N))]
                            ).reshape(NW, nch, CH)
    dummy = (jnp.arange(NW * BLK * CH, dtype=jnp.int32) % 256
             ).reshape(NW, BLK, CH)
    srcd_r = jnp.concatenate([src_r, dummy], axis=1)

    nchd = -(-E // (NW * DEGCH))
    epd = NW * DEGCH * nchd - E
    padd = jnp.arange(epd, dtype=jnp.int32)
    dstd_r = jnp.concatenate([dst, N + (padd % (NPAD - N))]
                             ).reshape(NW, nchd, DEGCH)

    zeros16 = jnp.zeros((RPT, 16), jnp.float32)
    ones16 = jnp.ones((DEGCH, 16), jnp.float32)
    zeros128 = jnp.zeros((CH, D), jnp.float32)

    deg = _deg_call(dstd_r, zeros16, ones16)
    hs1 = _mm_call(x, W1, deg)
    a1 = _agg_call(hs1, srcd_r, dst_r, zeros128)
    hs2 = _postmm_call(a1, deg, b1.reshape(1, D), W2)
    a2 = _agg_call(hs2, srcd_r, dst_r, zeros128)
    z2, p = _posthead_call(a2, deg, b2.reshape(1, D), Wp1,
                           bp1.reshape(1, D), prelu_a.reshape(1, 1),
                           Wp2, bp2.reshape(1, D))
    return (z2, p)


# speedup vs baseline: 1.3456x; 1.0243x over previous
import jax
import jax.numpy as jnp
from jax import lax
from jax.experimental import pallas as pl
from jax.experimental.pallas import tpu as pltpu
from jax.experimental.pallas import tpu_sc as plsc

N = 10000
NPAD = 10112
D = 128
NC = 2
NS = 16
NW = NC * NS
CH = 128
BLK = 16
GSPLIT = 4
DEGCH = 128
RPT = NPAD // NS
RT = 5000
STAGE = [128, 128, 128, 128, 120]


def _sc_mesh():
    return plsc.VectorSubcoreMesh(core_axis_name="c", subcore_axis_name="s")


def _deg_body(dst_hbm, zeros_hbm, ones_hbm, deg_hbm, idx_v, ones_v, stage_v,
              acc_sh):
    c = lax.axis_index("c")
    s = lax.axis_index("s")
    nch = dst_hbm.shape[1]
    w = c * NS + s

    pltpu.sync_copy(zeros_hbm, stage_v)
    pltpu.sync_copy(stage_v, acc_sh.at[pl.ds(s * RPT, RPT)])
    pltpu.sync_copy(ones_hbm, ones_v)
    pltpu.sync_copy(dst_hbm.at[w], idx_v)
    plsc.subcore_barrier()

    def body(j, carry):
        pltpu.sync_copy(ones_v, acc_sh.at[idx_v.at[j]], add=True)
        return carry

    lax.fori_loop(0, nch, body, 0)
    plsc.subcore_barrier()
    pltpu.sync_copy(acc_sh.at[pl.ds(s * RPT, RPT)], stage_v)
    pltpu.sync_copy(stage_v, deg_hbm.at[c, pl.ds(s * RPT, RPT)])


def _deg_call(dst_r, zeros16, ones16):
    nch = dst_r.shape[1]
    return pl.kernel(
        _deg_body,
        out_type=jax.ShapeDtypeStruct((NC, NPAD, 16), jnp.float32),
        mesh=_sc_mesh(),
        scratch_types=[
            pltpu.VMEM((nch, DEGCH), jnp.int32),
            pltpu.VMEM((DEGCH, 16), jnp.float32),
            pltpu.VMEM((RPT, 16), jnp.float32),
            pltpu.VMEM_SHARED((NPAD, 16), jnp.float32),
        ],
        compiler_params=pltpu.CompilerParams(use_tc_tiling_on_sc=False),
    )(dst_r, zeros16, ones16)


def _agg_body(hs_hbm, src_hbm, dst_hbm, zeros_hbm, out_hbm, rsrc_v, rdst_v,
              gb0, gb1, gs0, gs1, ss0, ss1, rs0, rs1, rd0, rd1, acc_sh):
    c = lax.axis_index("c")
    s = lax.axis_index("s")
    nch = dst_hbm.shape[1]
    nblk = nch // BLK
    w = c * NS + s
    r0 = s * RPT

    gbufs2 = (gb0, gb1)
    gsems2 = (gs0, gs1)
    ssems2 = (ss0, ss1)
    offs = []
    off = 0
    for sz in STAGE:
        offs.append((off, sz))
        off += sz

    def h_copy(k):
        off, sz = offs[k]
        b = k % 2

        @pl.when(c == 0)
        def _():
            pltpu.async_copy(hs_hbm.at[pl.ds(r0 + off, sz)],
                             gbufs2[b].at[pl.ds(0, sz)], gsems2[b])

        @pl.when(c != 0)
        def _():
            pltpu.async_copy(zeros_hbm.at[pl.ds(0, sz)],
                             gbufs2[b].at[pl.ds(0, sz)], gsems2[b])

    def h_wait(k):
        off, sz = offs[k]
        b = k % 2
        pltpu.make_async_copy(zeros_hbm.at[pl.ds(0, sz)],
                              gbufs2[b].at[pl.ds(0, sz)], gsems2[b]).wait()

    def a_copy(k, start=True):
        off, sz = offs[k]
        b = k % 2
        cp = pltpu.make_async_copy(gbufs2[b].at[pl.ds(0, sz)],
                                   acc_sh.at[pl.ds(r0 + off, sz)], ssems2[b])
        cp.start() if start else cp.wait()

    h_copy(0)
    for k in range(len(STAGE)):
        h_wait(k)
        a_copy(k)
        if k + 1 < len(STAGE):
            if k >= 1:
                a_copy(k - 1, start=False)
            h_copy(k + 1)
    a_copy(len(STAGE) - 2, start=False)
    a_copy(len(STAGE) - 1, start=False)

    rsems = (rs0, rs1)
    dsems = (rd0, rd1)

    def refill_src(blk, start=True):
        h = blk % 2
        cp = pltpu.make_async_copy(src_hbm.at[w, pl.ds(blk * BLK, BLK)],
                                   rsrc_v.at[h], rsems[h])
        cp.start() if start else cp.wait()

    def refill_dst(blk, start=True):
        h = blk % 2
        cp = pltpu.make_async_copy(dst_hbm.at[w, pl.ds(blk * BLK, BLK)],
                                   rdst_v.at[h], dsems[h])
        cp.start() if start else cp.wait()

    pltpu.sync_copy(src_hbm.at[w, pl.ds(0, BLK)], rsrc_v.at[0])
    pltpu.sync_copy(dst_hbm.at[w, pl.ds(0, BLK)], rdst_v.at[0])
    refill_src(1)
    refill_dst(1)
    plsc.subcore_barrier()

    QC = CH // GSPLIT

    def g_start(h, r, buf, sem):
        for q in range(GSPLIT):
            pltpu.async_copy(hs_hbm.at[rsrc_v.at[h, r, pl.ds(q * QC, QC)]],
                             buf.at[pl.ds(q * QC, QC)], sem)

    def g_wait(h, r, buf, sem):
        for q in range(GSPLIT):
            pltpu.make_async_copy(
                hs_hbm.at[rsrc_v.at[h, r, pl.ds(q * QC, QC)]],
                buf.at[pl.ds(q * QC, QC)], sem).wait()

    def s_start(h, r, buf, sem):
        pltpu.async_copy(buf, acc_sh.at[rdst_v.at[h, r]], sem, add=True)

    def s_wait(h, r, buf, sem):
        pltpu.make_async_copy(buf, acc_sh.at[rdst_v.at[h, r]], sem).wait()

    g_start(0, 0, gb0, gs0)
    g_start(0, 1, gb1, gs1)

    for bi in range(nblk):
        h = bi % 2
        hn = (bi + 1) % 2
        refill_src(bi + 1, start=False)
        if bi + 1 < nblk:
            refill_dst(bi + 1, start=False)

        def pair(kk, carry):
            r = 2 * kk
            g_wait(h, r, gb0, gs0)
            s_start(h, r, gb0, ss0)
            g_wait(h, r + 1, gb1, gs1)
            s_start(h, r + 1, gb1, ss1)
            s_wait(h, r, gb0, ss0)
            g_start(h, r + 2, gb0, gs0)
            s_wait(h, r + 1, gb1, ss1)
            g_start(h, r + 3, gb1, gs1)
            return carry

        lax.fori_loop(0, BLK // 2 - 1, pair, 0)
        r = BLK - 2
        g_wait(h, r, gb0, gs0)
        s_start(h, r, gb0, ss0)
        g_wait(h, r + 1, gb1, gs1)
        s_start(h, r + 1, gb1, ss1)
        s_wait(h, r, gb0, ss0)
        g_start(hn, 0, gb0, gs0)
        s_wait(h, r + 1, gb1, ss1)
        g_start(hn, 1, gb1, gs1)

        if bi + 2 <= nblk:
            refill_src(bi + 2)
        if bi + 2 < nblk:
            refill_dst(bi + 2)

    g_wait(nblk % 2, 0, gb0, gs0)
    g_wait(nblk % 2, 1, gb1, gs1)

    plsc.subcore_barrier()

    def d_copy(k):
        off, sz = offs[k]
        b = k % 2
        pltpu.async_copy(acc_sh.at[pl.ds(r0 + off, sz)],
                         gbufs2[b].at[pl.ds(0, sz)], gsems2[b])

    def d_wait(k):
        off, sz = offs[k]
        b = k % 2
        pltpu.make_async_copy(acc_sh.at[pl.ds(r0 + off, sz)],
                              gbufs2[b].at[pl.ds(0, sz)], gsems2[b]).wait()

    def o_copy(k, start=True):
        off, sz = offs[k]
        b = k % 2
        cp = pltpu.make_async_copy(gbufs2[b].at[pl.ds(0, sz)],
                                   out_hbm.at[c, pl.ds(r0 + off, sz)],
                                   ssems2[b])
        cp.start() if start else cp.wait()

    d_copy(0)
    for k in range(len(STAGE)):
        d_wait(k)
        o_copy(k)
        if k + 1 < len(STAGE):
            if k >= 1:
                o_copy(k - 1, start=False)
            d_copy(k + 1)
    o_copy(len(STAGE) - 2, start=False)
    o_copy(len(STAGE) - 1, start=False)


def _agg_call(hs, srcd_r, dst_r, zeros128):
    return pl.kernel(
        _agg_body,
        out_type=jax.ShapeDtypeStruct((NC, NPAD, D), jnp.float32),
        mesh=_sc_mesh(),
        scratch_types=[
            pltpu.VMEM((2, BLK, CH), jnp.int32),
            pltpu.VMEM((2, BLK, CH), jnp.int32),
            pltpu.VMEM((CH, D), jnp.float32),
            pltpu.VMEM((CH, D), jnp.float32),
            pltpu.SemaphoreType.DMA,
            pltpu.SemaphoreType.DMA,
            pltpu.SemaphoreType.DMA,
            pltpu.SemaphoreType.DMA,
            pltpu.SemaphoreType.DMA,
            pltpu.SemaphoreType.DMA,
            pltpu.SemaphoreType.DMA,
            pltpu.SemaphoreType.DMA,
            pltpu.VMEM_SHARED((NPAD, D), jnp.float32),
        ],
    )(hs, srcd_r, dst_r, zeros128)


def _mm_body(x_ref, w_ref, deg_ref, o_ref):
    deg = deg_ref[0, :, 0:1] + deg_ref[1, :, 0:1]
    dinv = lax.rsqrt(deg + 1.0)
    o_ref[...] = dinv * jnp.dot(x_ref[...], w_ref[...],
                                preferred_element_type=jnp.float32)


def _mm_call(x, w, deg):
    return pl.pallas_call(
        _mm_body,
        grid=(N // RT,),
        in_specs=[
            pl.BlockSpec((RT, D), lambda i: (i, 0)),
            pl.BlockSpec((D, D), lambda i: (0, 0)),
            pl.BlockSpec((NC, RT, 16), lambda i: (0, i, 0)),
        ],
        out_specs=pl.BlockSpec((RT, D), lambda i: (i, 0)),
        out_shape=jax.ShapeDtypeStruct((NPAD, D), jnp.float32),
    )(x, w, deg)


def _postmm_body(a_ref, deg_ref, b_ref, w_ref, o_ref):
    deg = deg_ref[0, :, 0:1] + deg_ref[1, :, 0:1]
    dinv = lax.rsqrt(deg + 1.0)
    z = jnp.maximum(dinv * (a_ref[0] + a_ref[1]) + b_ref[...], 0.0)
    o_ref[...] = dinv * jnp.dot(z, w_ref[...],
                                preferred_element_type=jnp.float32)


def _postmm_call(a, deg, b, w):
    return pl.pallas_call(
        _postmm_body,
        grid=(N // RT,),
        in_specs=[
            pl.BlockSpec((NC, RT, D), lambda i: (0, i, 0)),
            pl.BlockSpec((NC, RT, 16), lambda i: (0, i, 0)),
            pl.BlockSpec((1, D), lambda i: (0, 0)),
            pl.BlockSpec((D, D), lambda i: (0, 0)),
        ],
        out_specs=pl.BlockSpec((RT, D), lambda i: (i, 0)),
        out_shape=jax.ShapeDtypeStruct((NPAD, D), jnp.float32),
    )(a, deg, b, w)


def _posthead_body(a_ref, deg_ref, b_ref, w1_ref, b1_ref, pa_ref, w2_ref,
                   b2_ref, z_ref, p_ref):
    deg = deg_ref[0, :, 0:1] + deg_ref[1, :, 0:1]
    dinv = lax.rsqrt(deg + 1.0)
    z = jnp.maximum(dinv * (a_ref[0] + a_ref[1]) + b_ref[...], 0.0)
    z_ref[...] = z
    p = jnp.dot(z, w1_ref[...],
                preferred_element_type=jnp.float32) + b1_ref[...]
    p = jnp.where(p > 0, p, pa_ref[0, 0] * p)
    p_ref[...] = jnp.dot(p, w2_ref[...],
                         preferred_element_type=jnp.float32) + b2_ref[...]


def _posthead_call(a, deg, b, w1, b1, pa, w2, b2):
    full = lambda i: (0, 0)
    return pl.pallas_call(
        _posthead_body,
        grid=(N // RT,),
        in_specs=[
            pl.BlockSpec((NC, RT, D), lambda i: (0, i, 0)),
            pl.BlockSpec((NC, RT, 16), lambda i: (0, i, 0)),
            pl.BlockSpec((1, D), full),
            pl.BlockSpec((D, D), full),
            pl.BlockSpec((1, D), full),
            pl.BlockSpec((1, 1), full),
            pl.BlockSpec((D, D), full),
            pl.BlockSpec((1, D), full),
        ],
        out_specs=(
            pl.BlockSpec((RT, D), lambda i: (i, 0)),
            pl.BlockSpec((RT, D), lambda i: (i, 0)),
        ),
        out_shape=(
            jax.ShapeDtypeStruct((N, D), jnp.float32),
            jax.ShapeDtypeStruct((N, D), jnp.float32),
        ),
    )(a, deg, b, w1, b1, pa, w2, b2)


def kernel(x, edge_index, W1, b1, W2, b2, Wp1, bp1, prelu_a, Wp2, bp2):
    E = edge_index.shape[1]
    src = edge_index[0]
    dst = edge_index[1]

    nch = -(-E // (NW * CH))
    nch = -(-nch // BLK) * BLK
    ep = NW * CH * nch - E
    pad_ids = jnp.arange(ep, dtype=jnp.int32)
    src_r = jnp.concatenate([src, pad_ids % 256]).reshape(NW, nch, CH)
    dst_r = jnp.concatenate([dst, N + (pad_ids % (NPAD - N))]
                            ).reshape(NW, nch, CH)
    dummy = (jnp.arange(NW * BLK * CH, dtype=jnp.int32) % 256
             ).reshape(NW, BLK, CH)
    srcd_r = jnp.concatenate([src_r, dummy], axis=1)

    nchd = -(-E // (NW * DEGCH))
    epd = NW * DEGCH * nchd - E
    padd = jnp.arange(epd, dtype=jnp.int32)
    dstd_r = jnp.concatenate([dst, N + (padd % (NPAD - N))]
                             ).reshape(NW, nchd, DEGCH)

    zeros16 = jnp.zeros((RPT, 16), jnp.float32)
    ones16 = jnp.ones((DEGCH, 16), jnp.float32)
    zeros128 = jnp.zeros((CH, D), jnp.float32)

    deg = _deg_call(dstd_r, zeros16, ones16)
    hs1 = _mm_call(x, W1, deg)
    a1 = _agg_call(hs1, srcd_r, dst_r, zeros128)
    hs2 = _postmm_call(a1, deg, b1.reshape(1, D), W2)
    a2 = _agg_call(hs2, srcd_r, dst_r, zeros128)
    z2, p = _posthead_call(a2, deg, b2.reshape(1, D), Wp1,
                           bp1.reshape(1, D), prelu_a.reshape(1, 1),
                           Wp2, bp2.reshape(1, D))
    return (z2, p)


# speedup vs baseline: 1.5770x; 1.1720x over previous
import jax
import jax.numpy as jnp
from jax import lax
from jax.experimental import pallas as pl
from jax.experimental.pallas import tpu as pltpu
from jax.experimental.pallas import tpu_sc as plsc

N = 10000
NPAD = 10112
D = 128
NC = 2
NS = 16
NW = NC * NS
CH = 128
BLK = 16
GSPLIT = 4
DEGCH = 128
RPT = NPAD // NS
RT = 5000
STAGE = [128, 128, 128, 128, 120]


def _sc_mesh():
    return plsc.VectorSubcoreMesh(core_axis_name="c", subcore_axis_name="s")


def _deg_body(dst_hbm, zeros_hbm, ones_hbm, deg_hbm, idx_v, ones_v, stage_v,
              acc_sh):
    c = lax.axis_index("c")
    s = lax.axis_index("s")
    nch = dst_hbm.shape[1]
    w = c * NS + s

    pltpu.sync_copy(zeros_hbm, stage_v)
    pltpu.sync_copy(stage_v, acc_sh.at[pl.ds(s * RPT, RPT)])
    pltpu.sync_copy(ones_hbm, ones_v)
    pltpu.sync_copy(dst_hbm.at[w], idx_v)
    plsc.subcore_barrier()

    def body(j, carry):
        pltpu.sync_copy(ones_v, acc_sh.at[idx_v.at[j]], add=True)
        return carry

    lax.fori_loop(0, nch, body, 0)
    plsc.subcore_barrier()
    pltpu.sync_copy(acc_sh.at[pl.ds(s * RPT, RPT)], stage_v)
    pltpu.sync_copy(stage_v, deg_hbm.at[c, pl.ds(s * RPT, RPT)])


def _deg_call(dst_r, zeros16, ones16):
    nch = dst_r.shape[1]
    return pl.kernel(
        _deg_body,
        out_type=jax.ShapeDtypeStruct((NC, NPAD, 16), jnp.float32),
        mesh=_sc_mesh(),
        scratch_types=[
            pltpu.VMEM((nch, DEGCH), jnp.int32),
            pltpu.VMEM((DEGCH, 16), jnp.float32),
            pltpu.VMEM((RPT, 16), jnp.float32),
            pltpu.VMEM_SHARED((NPAD, 16), jnp.float32),
        ],
        compiler_params=pltpu.CompilerParams(use_tc_tiling_on_sc=False),
    )(dst_r, zeros16, ones16)


def _agg_body(hs_hbm, src_hbm, dst_hbm, zeros_hbm, out_hbm, rsrc_v, rdst_v,
              gb0, gb1, gs0, gs1, ss0, ss1, rs0, rs1, rd0, rd1, acc_sh):
    c = lax.axis_index("c")
    s = lax.axis_index("s")
    nch = dst_hbm.shape[1]
    nblk = nch // BLK
    w = c * NS + s
    r0 = s * RPT

    gbufs2 = (gb0, gb1)
    gsems2 = (gs0, gs1)
    ssems2 = (ss0, ss1)
    offs = []
    off = 0
    for sz in STAGE:
        offs.append((off, sz))
        off += sz

    def h_copy(k):
        off, sz = offs[k]
        b = k % 2

        @pl.when(c == 0)
        def _():
            pltpu.async_copy(hs_hbm.at[pl.ds(r0 + off, sz)],
                             gbufs2[b].at[pl.ds(0, sz)], gsems2[b])

        @pl.when(c != 0)
        def _():
            pltpu.async_copy(zeros_hbm.at[pl.ds(0, sz)],
                             gbufs2[b].at[pl.ds(0, sz)], gsems2[b])

    def h_wait(k):
        off, sz = offs[k]
        b = k % 2
        pltpu.make_async_copy(zeros_hbm.at[pl.ds(0, sz)],
                              gbufs2[b].at[pl.ds(0, sz)], gsems2[b]).wait()

    def a_copy(k, start=True):
        off, sz = offs[k]
        b = k % 2
        cp = pltpu.make_async_copy(gbufs2[b].at[pl.ds(0, sz)],
                                   acc_sh.at[pl.ds(r0 + off, sz)], ssems2[b])
        cp.start() if start else cp.wait()

    h_copy(0)
    for k in range(len(STAGE)):
        h_wait(k)
        a_copy(k)
        if k + 1 < len(STAGE):
            if k >= 1:
                a_copy(k - 1, start=False)
            h_copy(k + 1)
    a_copy(len(STAGE) - 2, start=False)
    a_copy(len(STAGE) - 1, start=False)

    rsems = (rs0, rs1)
    dsems = (rd0, rd1)

    def refill_src(blk, start=True):
        h = blk % 2
        cp = pltpu.make_async_copy(src_hbm.at[w, pl.ds(blk * BLK, BLK)],
                                   rsrc_v.at[h], rsems[h])
        cp.start() if start else cp.wait()

    def refill_dst(blk, start=True):
        h = blk % 2
        cp = pltpu.make_async_copy(dst_hbm.at[w, pl.ds(blk * BLK, BLK)],
                                   rdst_v.at[h], dsems[h])
        cp.start() if start else cp.wait()

    pltpu.sync_copy(src_hbm.at[w, pl.ds(0, BLK)], rsrc_v.at[0])
    pltpu.sync_copy(dst_hbm.at[w, pl.ds(0, BLK)], rdst_v.at[0])
    refill_src(1)
    refill_dst(1)
    plsc.subcore_barrier()

    QC = CH // GSPLIT

    def g_start(h, r, buf, sem):
        for q in range(GSPLIT):
            pltpu.async_copy(hs_hbm.at[rsrc_v.at[h, r, pl.ds(q * QC, QC)]],
                             buf.at[pl.ds(q * QC, QC)], sem)

    def g_wait(h, r, buf, sem):
        for q in range(GSPLIT):
            pltpu.make_async_copy(
                hs_hbm.at[rsrc_v.at[h, r, pl.ds(q * QC, QC)]],
                buf.at[pl.ds(q * QC, QC)], sem).wait()

    def s_start(h, r, buf, sem):
        pltpu.async_copy(buf, acc_sh.at[rdst_v.at[h, r]], sem, add=True)

    def s_wait(h, r, buf, sem):
        pltpu.make_async_copy(buf, acc_sh.at[rdst_v.at[h, r]], sem).wait()

    g_start(0, 0, gb0, gs0)
    g_start(0, 1, gb1, gs1)

    for bi in range(nblk):
        h = bi % 2
        hn = (bi + 1) % 2
        refill_src(bi + 1, start=False)
        if bi + 1 < nblk:
            refill_dst(bi + 1, start=False)

        def pair(kk, carry):
            r = 2 * kk
            g_wait(h, r, gb0, gs0)
            s_start(h, r, gb0, ss0)
            g_wait(h, r + 1, gb1, gs1)
            s_start(h, r + 1, gb1, ss1)
            s_wait(h, r, gb0, ss0)
            g_start(h, r + 2, gb0, gs0)
            s_wait(h, r + 1, gb1, ss1)
            g_start(h, r + 3, gb1, gs1)
            return carry

        lax.fori_loop(0, BLK // 2 - 1, pair, 0)
        r = BLK - 2
        g_wait(h, r, gb0, gs0)
        s_start(h, r, gb0, ss0)
        g_wait(h, r + 1, gb1, gs1)
        s_start(h, r + 1, gb1, ss1)
        s_wait(h, r, gb0, ss0)
        g_start(hn, 0, gb0, gs0)
        s_wait(h, r + 1, gb1, ss1)
        g_start(hn, 1, gb1, gs1)

        if bi + 2 <= nblk:
            refill_src(bi + 2)
        if bi + 2 < nblk:
            refill_dst(bi + 2)

    g_wait(nblk % 2, 0, gb0, gs0)
    g_wait(nblk % 2, 1, gb1, gs1)

    plsc.subcore_barrier()

    def d_copy(k):
        off, sz = offs[k]
        b = k % 2
        pltpu.async_copy(acc_sh.at[pl.ds(r0 + off, sz)],
                         gbufs2[b].at[pl.ds(0, sz)], gsems2[b])

    def d_wait(k):
        off, sz = offs[k]
        b = k % 2
        pltpu.make_async_copy(acc_sh.at[pl.ds(r0 + off, sz)],
                              gbufs2[b].at[pl.ds(0, sz)], gsems2[b]).wait()

    def o_copy(k, start=True):
        off, sz = offs[k]
        b = k % 2
        cp = pltpu.make_async_copy(gbufs2[b].at[pl.ds(0, sz)],
                                   out_hbm.at[c, pl.ds(r0 + off, sz)],
                                   ssems2[b])
        cp.start() if start else cp.wait()

    d_copy(0)
    for k in range(len(STAGE)):
        d_wait(k)
        o_copy(k)
        if k + 1 < len(STAGE):
            if k >= 1:
                o_copy(k - 1, start=False)
            d_copy(k + 1)
    o_copy(len(STAGE) - 2, start=False)
    o_copy(len(STAGE) - 1, start=False)


def _agg_call(hs, srcd_r, dst_r, zeros128):
    return pl.kernel(
        _agg_body,
        out_type=jax.ShapeDtypeStruct((NC, NPAD, D), jnp.bfloat16),
        mesh=_sc_mesh(),
        scratch_types=[
            pltpu.VMEM((2, BLK, CH), jnp.int32),
            pltpu.VMEM((2, BLK, CH), jnp.int32),
            pltpu.VMEM((CH, D), jnp.bfloat16),
            pltpu.VMEM((CH, D), jnp.bfloat16),
            pltpu.SemaphoreType.DMA,
            pltpu.SemaphoreType.DMA,
            pltpu.SemaphoreType.DMA,
            pltpu.SemaphoreType.DMA,
            pltpu.SemaphoreType.DMA,
            pltpu.SemaphoreType.DMA,
            pltpu.SemaphoreType.DMA,
            pltpu.SemaphoreType.DMA,
            pltpu.VMEM_SHARED((NPAD, D), jnp.bfloat16),
        ],
        compiler_params=pltpu.CompilerParams(use_tc_tiling_on_sc=False),
    )(hs, srcd_r, dst_r, zeros128)


def _mm_body(x_ref, w_ref, deg_ref, o_ref):
    deg = deg_ref[0, :, 0:1] + deg_ref[1, :, 0:1]
    dinv = lax.rsqrt(deg + 1.0)
    o_ref[...] = (dinv * jnp.dot(x_ref[...], w_ref[...],
                                 preferred_element_type=jnp.float32)
                  ).astype(jnp.bfloat16)


def _mm_call(x, w, deg):
    return pl.pallas_call(
        _mm_body,
        grid=(N // RT,),
        in_specs=[
            pl.BlockSpec((RT, D), lambda i: (i, 0)),
            pl.BlockSpec((D, D), lambda i: (0, 0)),
            pl.BlockSpec((NC, RT, 16), lambda i: (0, i, 0)),
        ],
        out_specs=pl.BlockSpec((RT, D), lambda i: (i, 0)),
        out_shape=jax.ShapeDtypeStruct((NPAD, D), jnp.bfloat16),
    )(x, w, deg)


def _postmm_body(a_ref, deg_ref, b_ref, w_ref, o_ref):
    deg = deg_ref[0, :, 0:1] + deg_ref[1, :, 0:1]
    dinv = lax.rsqrt(deg + 1.0)
    asum = a_ref[0].astype(jnp.float32) + a_ref[1].astype(jnp.float32)
    z = jnp.maximum(dinv * asum + b_ref[...], 0.0)
    o_ref[...] = (dinv * jnp.dot(z, w_ref[...],
                                 preferred_element_type=jnp.float32)
                  ).astype(jnp.bfloat16)


def _postmm_call(a, deg, b, w):
    return pl.pallas_call(
        _postmm_body,
        grid=(N // RT,),
        in_specs=[
            pl.BlockSpec((NC, RT, D), lambda i: (0, i, 0)),
            pl.BlockSpec((NC, RT, 16), lambda i: (0, i, 0)),
            pl.BlockSpec((1, D), lambda i: (0, 0)),
            pl.BlockSpec((D, D), lambda i: (0, 0)),
        ],
        out_specs=pl.BlockSpec((RT, D), lambda i: (i, 0)),
        out_shape=jax.ShapeDtypeStruct((NPAD, D), jnp.bfloat16),
    )(a, deg, b, w)


def _posthead_body(a_ref, deg_ref, b_ref, w1_ref, b1_ref, pa_ref, w2_ref,
                   b2_ref, z_ref, p_ref):
    deg = deg_ref[0, :, 0:1] + deg_ref[1, :, 0:1]
    dinv = lax.rsqrt(deg + 1.0)
    asum = a_ref[0].astype(jnp.float32) + a_ref[1].astype(jnp.float32)
    z = jnp.maximum(dinv * asum + b_ref[...], 0.0)
    z_ref[...] = z
    p = jnp.dot(z, w1_ref[...],
                preferred_element_type=jnp.float32) + b1_ref[...]
    p = jnp.where(p > 0, p, pa_ref[0, 0] * p)
    p_ref[...] = jnp.dot(p, w2_ref[...],
                         preferred_element_type=jnp.float32) + b2_ref[...]


def _posthead_call(a, deg, b, w1, b1, pa, w2, b2):
    full = lambda i: (0, 0)
    return pl.pallas_call(
        _posthead_body,
        grid=(N // RT,),
        in_specs=[
            pl.BlockSpec((NC, RT, D), lambda i: (0, i, 0)),
            pl.BlockSpec((NC, RT, 16), lambda i: (0, i, 0)),
            pl.BlockSpec((1, D), full),
            pl.BlockSpec((D, D), full),
            pl.BlockSpec((1, D), full),
            pl.BlockSpec((1, 1), full),
            pl.BlockSpec((D, D), full),
            pl.BlockSpec((1, D), full),
        ],
        out_specs=(
            pl.BlockSpec((RT, D), lambda i: (i, 0)),
            pl.BlockSpec((RT, D), lambda i: (i, 0)),
        ),
        out_shape=(
            jax.ShapeDtypeStruct((N, D), jnp.float32),
            jax.ShapeDtypeStruct((N, D), jnp.float32),
        ),
    )(a, deg, b, w1, b1, pa, w2, b2)


def kernel(x, edge_index, W1, b1, W2, b2, Wp1, bp1, prelu_a, Wp2, bp2):
    E = edge_index.shape[1]
    src = edge_index[0]
    dst = edge_index[1]

    nch = -(-E // (NW * CH))
    nch = -(-nch // BLK) * BLK
    ep = NW * CH * nch - E
    pad_ids = jnp.arange(ep, dtype=jnp.int32)
    src_r = jnp.concatenate([src, pad_ids % 256]).reshape(NW, nch, CH)
    dst_r = jnp.concatenate([dst, N + (pad_ids % (NPAD - N))]
                            ).reshape(NW, nch, CH)
    dummy = (jnp.arange(NW * BLK * CH, dtype=jnp.int32) % 256
             ).reshape(NW, BLK, CH)
    srcd_r = jnp.concatenate([src_r, dummy], axis=1)

    nchd = -(-E // (NW * DEGCH))
    epd = NW * DEGCH * nchd - E
    padd = jnp.arange(epd, dtype=jnp.int32)
    dstd_r = jnp.concatenate([dst, N + (padd % (NPAD - N))]
                             ).reshape(NW, nchd, DEGCH)

    zeros16 = jnp.zeros((RPT, 16), jnp.float32)
    ones16 = jnp.ones((DEGCH, 16), jnp.float32)
    zeros128 = jnp.zeros((CH, D), jnp.bfloat16)

    deg = _deg_call(dstd_r, zeros16, ones16)
    hs1 = _mm_call(x, W1, deg)
    a1 = _agg_call(hs1, srcd_r, dst_r, zeros128)
    hs2 = _postmm_call(a1, deg, b1.reshape(1, D), W2)
    a2 = _agg_call(hs2, srcd_r, dst_r, zeros128)
    z2, p = _posthead_call(a2, deg, b2.reshape(1, D), Wp1,
                           bp1.reshape(1, D), prelu_a.reshape(1, 1),
                           Wp2, bp2.reshape(1, D))
    return (z2, p)


# speedup vs baseline: 1.8158x; 1.1514x over previous
import jax
import jax.numpy as jnp
from jax import lax
from jax.experimental import pallas as pl
from jax.experimental.pallas import tpu as pltpu
from jax.experimental.pallas import tpu_sc as plsc

N = 10000
NPAD = 10112
D = 128
NC = 2
NS = 16
NW = NC * NS
CH = 128
BLK = 16
GSPLIT = 2
DEGCH = 128
RPT = NPAD // NS
RT = 5000
STAGE = [128, 128, 128, 128, 120]


def _sc_mesh():
    return plsc.VectorSubcoreMesh(core_axis_name="c", subcore_axis_name="s")


def _deg_body(dst_hbm, zeros_hbm, ones_hbm, deg_hbm, idx_v, ones_v, stage_v,
              acc_sh):
    c = lax.axis_index("c")
    s = lax.axis_index("s")
    nch = dst_hbm.shape[1]
    w = c * NS + s

    pltpu.sync_copy(zeros_hbm, stage_v)
    pltpu.sync_copy(stage_v, acc_sh.at[pl.ds(s * RPT, RPT)])
    pltpu.sync_copy(ones_hbm, ones_v)
    pltpu.sync_copy(dst_hbm.at[w], idx_v)
    plsc.subcore_barrier()

    def body(j, carry):
        pltpu.sync_copy(ones_v, acc_sh.at[idx_v.at[j]], add=True)
        return carry

    lax.fori_loop(0, nch, body, 0)
    plsc.subcore_barrier()
    pltpu.sync_copy(acc_sh.at[pl.ds(s * RPT, RPT)], stage_v)
    pltpu.sync_copy(stage_v, deg_hbm.at[c, pl.ds(s * RPT, RPT)])


def _deg_call(dst_r, zeros16, ones16):
    nch = dst_r.shape[1]
    return pl.kernel(
        _deg_body,
        out_type=jax.ShapeDtypeStruct((NC, NPAD, 16), jnp.float32),
        mesh=_sc_mesh(),
        scratch_types=[
            pltpu.VMEM((nch, DEGCH), jnp.int32),
            pltpu.VMEM((DEGCH, 16), jnp.float32),
            pltpu.VMEM((RPT, 16), jnp.float32),
            pltpu.VMEM_SHARED((NPAD, 16), jnp.float32),
        ],
        compiler_params=pltpu.CompilerParams(use_tc_tiling_on_sc=False),
    )(dst_r, zeros16, ones16)


def _agg_body(hs_hbm, src_hbm, dst_hbm, zeros_hbm, out_hbm, rsrc_v, rdst_v,
              gb0, gb1, gb2, gb3, gs0, gs1, gs2, gs3, ss0, ss1, ss2, ss3,
              rs0, rs1, rd0, rd1, acc_sh):
    c = lax.axis_index("c")
    s = lax.axis_index("s")
    nch = dst_hbm.shape[1]
    nblk = nch // BLK
    w = c * NS + s
    r0 = s * RPT

    gbufs2 = (gb0, gb1)
    gsems2 = (gs0, gs1)
    ssems2 = (ss0, ss1)
    offs = []
    off = 0
    for sz in STAGE:
        offs.append((off, sz))
        off += sz

    def h_copy(k):
        off, sz = offs[k]
        b = k % 2

        @pl.when(c == 0)
        def _():
            pltpu.async_copy(hs_hbm.at[pl.ds(r0 + off, sz)],
                             gbufs2[b].at[pl.ds(0, sz)], gsems2[b])

        @pl.when(c != 0)
        def _():
            pltpu.async_copy(zeros_hbm.at[pl.ds(0, sz)],
                             gbufs2[b].at[pl.ds(0, sz)], gsems2[b])

    def h_wait(k):
        off, sz = offs[k]
        b = k % 2
        pltpu.make_async_copy(zeros_hbm.at[pl.ds(0, sz)],
                              gbufs2[b].at[pl.ds(0, sz)], gsems2[b]).wait()

    def a_copy(k, start=True):
        off, sz = offs[k]
        b = k % 2
        cp = pltpu.make_async_copy(gbufs2[b].at[pl.ds(0, sz)],
                                   acc_sh.at[pl.ds(r0 + off, sz)], ssems2[b])
        cp.start() if start else cp.wait()

    h_copy(0)
    for k in range(len(STAGE)):
        h_wait(k)
        a_copy(k)
        if k + 1 < len(STAGE):
            if k >= 1:
                a_copy(k - 1, start=False)
            h_copy(k + 1)
    a_copy(len(STAGE) - 2, start=False)
    a_copy(len(STAGE) - 1, start=False)

    rsems = (rs0, rs1)
    dsems = (rd0, rd1)

    def refill_src(blk, start=True):
        h = blk % 2
        cp = pltpu.make_async_copy(src_hbm.at[w, pl.ds(blk * BLK, BLK)],
                                   rsrc_v.at[h], rsems[h])
        cp.start() if start else cp.wait()

    def refill_dst(blk, start=True):
        h = blk % 2
        cp = pltpu.make_async_copy(dst_hbm.at[w, pl.ds(blk * BLK, BLK)],
                                   rdst_v.at[h], dsems[h])
        cp.start() if start else cp.wait()

    pltpu.sync_copy(src_hbm.at[w, pl.ds(0, BLK)], rsrc_v.at[0])
    pltpu.sync_copy(dst_hbm.at[w, pl.ds(0, BLK)], rdst_v.at[0])
    refill_src(1)
    refill_dst(1)
    plsc.subcore_barrier()

    QC = CH // GSPLIT

    def g_start(h, r, buf, sem):
        for q in range(GSPLIT):
            pltpu.async_copy(hs_hbm.at[rsrc_v.at[h, r, pl.ds(q * QC, QC)]],
                             buf.at[pl.ds(q * QC, QC)], sem)

    def g_wait(h, r, buf, sem):
        for q in range(GSPLIT):
            pltpu.make_async_copy(
                hs_hbm.at[rsrc_v.at[h, r, pl.ds(q * QC, QC)]],
                buf.at[pl.ds(q * QC, QC)], sem).wait()

    def s_start(h, r, buf, sem):
        pltpu.async_copy(buf, acc_sh.at[rdst_v.at[h, r]], sem, add=True)

    def s_wait(h, r, buf, sem):
        pltpu.make_async_copy(buf, acc_sh.at[rdst_v.at[h, r]], sem).wait()

    bufs = (gb0, gb1, gb2, gb3)
    gsems = (gs0, gs1, gs2, gs3)
    ssems = (ss0, ss1, ss2, ss3)

    def gst(h, r, b):
        g_start(h, r, bufs[b], gsems[b])

    def gwt(h, r, b):
        g_wait(h, r, bufs[b], gsems[b])

    def sst(h, r, b):
        s_start(h, r, bufs[b], ssems[b])

    def swt(h, r, b):
        s_wait(h, r, bufs[b], ssems[b])

    for b in range(4):
        gst(0, b, b)

    for bi in range(nblk):
        h = bi % 2
        hn = (bi + 1) % 2
        refill_src(bi + 1, start=False)
        if bi + 1 < nblk:
            refill_dst(bi + 1, start=False)

        def group(kk, carry):
            r = 4 * kk
            for b in range(4):
                gwt(h, r + b, b)
                sst(h, r + b, b)
            for b in range(4):
                swt(h, r + b, b)
                gst(h, r + 4 + b, b)
            return carry

        lax.fori_loop(0, BLK // 4 - 1, group, 0)
        r = BLK - 4
        for b in range(4):
            gwt(h, r + b, b)
            sst(h, r + b, b)
        for b in range(4):
            swt(h, r + b, b)
            gst(hn, b, b)

        if bi + 2 <= nblk:
            refill_src(bi + 2)
        if bi + 2 < nblk:
            refill_dst(bi + 2)

    for b in range(4):
        gwt(nblk % 2, b, b)

    plsc.subcore_barrier()

    def d_copy(k):
        off, sz = offs[k]
        b = k % 2
        pltpu.async_copy(acc_sh.at[pl.ds(r0 + off, sz)],
                         gbufs2[b].at[pl.ds(0, sz)], gsems2[b])

    def d_wait(k):
        off, sz = offs[k]
        b = k % 2
        pltpu.make_async_copy(acc_sh.at[pl.ds(r0 + off, sz)],
                              gbufs2[b].at[pl.ds(0, sz)], gsems2[b]).wait()

    def o_copy(k, start=True):
        off, sz = offs[k]
        b = k % 2
        cp = pltpu.make_async_copy(gbufs2[b].at[pl.ds(0, sz)],
                                   out_hbm.at[c, pl.ds(r0 + off, sz)],
                                   ssems2[b])
        cp.start() if start else cp.wait()

    d_copy(0)
    for k in range(len(STAGE)):
        d_wait(k)
        o_copy(k)
        if k + 1 < len(STAGE):
            if k >= 1:
                o_copy(k - 1, start=False)
            d_copy(k + 1)
    o_copy(len(STAGE) - 2, start=False)
    o_copy(len(STAGE) - 1, start=False)


def _agg_call(hs, srcd_r, dst_r, zeros128):
    return pl.kernel(
        _agg_body,
        out_type=jax.ShapeDtypeStruct((NC, NPAD, D), jnp.bfloat16),
        mesh=_sc_mesh(),
        scratch_types=[
            pltpu.VMEM((2, BLK, CH), jnp.int32),
            pltpu.VMEM((2, BLK, CH), jnp.int32),
            pltpu.VMEM((CH, D), jnp.bfloat16),
            pltpu.VMEM((CH, D), jnp.bfloat16),
            pltpu.VMEM((CH, D), jnp.bfloat16),
            pltpu.VMEM((CH, D), jnp.bfloat16),
            pltpu.SemaphoreType.DMA,
            pltpu.SemaphoreType.DMA,
            pltpu.SemaphoreType.DMA,
            pltpu.SemaphoreType.DMA,
            pltpu.SemaphoreType.DMA,
            pltpu.SemaphoreType.DMA,
            pltpu.SemaphoreType.DMA,
            pltpu.SemaphoreType.DMA,
            pltpu.SemaphoreType.DMA,
            pltpu.SemaphoreType.DMA,
            pltpu.SemaphoreType.DMA,
            pltpu.SemaphoreType.DMA,
            pltpu.VMEM_SHARED((NPAD, D), jnp.bfloat16),
        ],
        compiler_params=pltpu.CompilerParams(use_tc_tiling_on_sc=False),
    )(hs, srcd_r, dst_r, zeros128)


def _mm_body(x_ref, w_ref, deg_ref, o_ref):
    deg = deg_ref[0, :, 0:1] + deg_ref[1, :, 0:1]
    dinv = lax.rsqrt(deg + 1.0)
    o_ref[...] = (dinv * jnp.dot(x_ref[...], w_ref[...],
                                 preferred_element_type=jnp.float32)
                  ).astype(jnp.bfloat16)


def _mm_call(x, w, deg):
    return pl.pallas_call(
        _mm_body,
        grid=(N // RT,),
        in_specs=[
            pl.BlockSpec((RT, D), lambda i: (i, 0)),
            pl.BlockSpec((D, D), lambda i: (0, 0)),
            pl.BlockSpec((NC, RT, 16), lambda i: (0, i, 0)),
        ],
        out_specs=pl.BlockSpec((RT, D), lambda i: (i, 0)),
        out_shape=jax.ShapeDtypeStruct((NPAD, D), jnp.bfloat16),
    )(x, w, deg)


def _postmm_body(a_ref, deg_ref, b_ref, w_ref, o_ref):
    deg = deg_ref[0, :, 0:1] + deg_ref[1, :, 0:1]
    dinv = lax.rsqrt(deg + 1.0)
    asum = a_ref[0].astype(jnp.float32) + a_ref[1].astype(jnp.float32)
    z = jnp.maximum(dinv * asum + b_ref[...], 0.0)
    o_ref[...] = (dinv * jnp.dot(z, w_ref[...],
                                 preferred_element_type=jnp.float32)
                  ).astype(jnp.bfloat16)


def _postmm_call(a, deg, b, w):
    return pl.pallas_call(
        _postmm_body,
        grid=(N // RT,),
        in_specs=[
            pl.BlockSpec((NC, RT, D), lambda i: (0, i, 0)),
            pl.BlockSpec((NC, RT, 16), lambda i: (0, i, 0)),
            pl.BlockSpec((1, D), lambda i: (0, 0)),
            pl.BlockSpec((D, D), lambda i: (0, 0)),
        ],
        out_specs=pl.BlockSpec((RT, D), lambda i: (i, 0)),
        out_shape=jax.ShapeDtypeStruct((NPAD, D), jnp.bfloat16),
    )(a, deg, b, w)


def _posthead_body(a_ref, deg_ref, b_ref, w1_ref, b1_ref, pa_ref, w2_ref,
                   b2_ref, z_ref, p_ref):
    deg = deg_ref[0, :, 0:1] + deg_ref[1, :, 0:1]
    dinv = lax.rsqrt(deg + 1.0)
    asum = a_ref[0].astype(jnp.float32) + a_ref[1].astype(jnp.float32)
    z = jnp.maximum(dinv * asum + b_ref[...], 0.0)
    z_ref[...] = z
    p = jnp.dot(z, w1_ref[...],
                preferred_element_type=jnp.float32) + b1_ref[...]
    p = jnp.where(p > 0, p, pa_ref[0, 0] * p)
    p_ref[...] = jnp.dot(p, w2_ref[...],
                         preferred_element_type=jnp.float32) + b2_ref[...]


def _posthead_call(a, deg, b, w1, b1, pa, w2, b2):
    full = lambda i: (0, 0)
    return pl.pallas_call(
        _posthead_body,
        grid=(N // RT,),
        in_specs=[
            pl.BlockSpec((NC, RT, D), lambda i: (0, i, 0)),
            pl.BlockSpec((NC, RT, 16), lambda i: (0, i, 0)),
            pl.BlockSpec((1, D), full),
            pl.BlockSpec((D, D), full),
            pl.BlockSpec((1, D), full),
            pl.BlockSpec((1, 1), full),
            pl.BlockSpec((D, D), full),
            pl.BlockSpec((1, D), full),
        ],
        out_specs=(
            pl.BlockSpec((RT, D), lambda i: (i, 0)),
            pl.BlockSpec((RT, D), lambda i: (i, 0)),
        ),
        out_shape=(
            jax.ShapeDtypeStruct((N, D), jnp.float32),
            jax.ShapeDtypeStruct((N, D), jnp.float32),
        ),
    )(a, deg, b, w1, b1, pa, w2, b2)


def kernel(x, edge_index, W1, b1, W2, b2, Wp1, bp1, prelu_a, Wp2, bp2):
    E = edge_index.shape[1]
    src = edge_index[0]
    dst = edge_index[1]

    nch = -(-E // (NW * CH))
    nch = -(-nch // BLK) * BLK
    ep = NW * CH * nch - E
    pad_ids = jnp.arange(ep, dtype=jnp.int32)
    src_r = jnp.concatenate([src, pad_ids % 256]).reshape(NW, nch, CH)
    dst_r = jnp.concatenate([dst, N + (pad_ids % (NPAD - N))]
                            ).reshape(NW, nch, CH)
    dummy = (jnp.arange(NW * BLK * CH, dtype=jnp.int32) % 256
             ).reshape(NW, BLK, CH)
    srcd_r = jnp.concatenate([src_r, dummy], axis=1)

    nchd = -(-E // (NW * DEGCH))
    epd = NW * DEGCH * nchd - E
    padd = jnp.arange(epd, dtype=jnp.int32)
    dstd_r = jnp.concatenate([dst, N + (padd % (NPAD - N))]
                             ).reshape(NW, nchd, DEGCH)

    zeros16 = jnp.zeros((RPT, 16), jnp.float32)
    ones16 = jnp.ones((DEGCH, 16), jnp.float32)
    zeros128 = jnp.zeros((CH, D), jnp.bfloat16)

    deg = _deg_call(dstd_r, zeros16, ones16)
    hs1 = _mm_call(x, W1, deg)
    a1 = _agg_call(hs1, srcd_r, dst_r, zeros128)
    hs2 = _postmm_call(a1, deg, b1.reshape(1, D), W2)
    a2 = _agg_call(hs2, srcd_r, dst_r, zeros128)
    z2, p = _posthead_call(a2, deg, b2.reshape(1, D), Wp1,
                           bp1.reshape(1, D), prelu_a.reshape(1, 1),
                           Wp2, bp2.reshape(1, D))
    return (z2, p)
